# baseline (speedup 1.0000x reference)
import math

import jax
import jax.numpy as jnp
from jax import lax
from jax.experimental import pallas as pl
from jax.experimental.pallas import tpu as pltpu
from jax.experimental.pallas import tpu_sc as plsc

N_TOK = 4096
N_CLS = 32000
PAD = 0
SMOOTHING = 0.1
CONF = 1.0 - SMOOTHING
EPS = SMOOTHING / (N_CLS - 2)
C0 = (N_CLS - 2) * EPS * math.log(EPS) + CONF * math.log(CONF)

TC_ROWS = 2048
SC_ROWS = N_TOK - TC_ROWS

ROW_BLK = 128


def _masked_sum_body(tgt_ref, x_ref, acc_ref):
    @pl.when(pl.program_id(0) == 0)
    def _():
        acc_ref[0, 0] = 0.0

    m = (tgt_ref[...] != PAD).astype(jnp.float32)
    acc_ref[0, 0] += jnp.sum(x_ref[...] * m)


_masked_sum = pl.pallas_call(
    _masked_sum_body,
    grid=(TC_ROWS // ROW_BLK,),
    in_specs=[
        pl.BlockSpec((ROW_BLK, 1), lambda i: (i, 0)),
        pl.BlockSpec((ROW_BLK, N_CLS), lambda i: (i, 0)),
    ],
    out_specs=pl.BlockSpec((1, 1), lambda i: (0, 0), memory_space=pltpu.SMEM),
    out_shape=jax.ShapeDtypeStruct((1, 1), jnp.float32),
)

L = 16
NC, NS = 2, 16
NW = NC * NS
BPW = N_TOK // NW
R_W = SC_ROWS // NW
N_GRP = R_W // 16
CW = 3200
N_CHK = N_CLS // CW


def _sc_body(x_hbm, xf_hbm, tgt_hbm, out_hbm,
             tgt_v, idx_v, idx0_v, vals_v, vals0_v, dtgt_v, buf0, buf1,
             res_v, semg, sem0, sem1):
    wid = lax.axis_index("s") * NC + lax.axis_index("c")
    base = wid * BPW
    lane_ids = lax.iota(jnp.int32, L)

    pltpu.sync_copy(tgt_hbm.at[pl.ds(base, BPW)], tgt_v)
    for j in range(BPW // L):
        t = tgt_v[pl.ds(j * L, L)]
        row_start = (base + j * L + lane_ids) * N_CLS
        idx_v[pl.ds(j * L, L)] = row_start + t
        idx0_v[pl.ds(j * L, L)] = row_start
    gcp = pltpu.async_copy(xf_hbm.at[idx_v], vals_v, semg)
    gcp0 = pltpu.async_copy(xf_hbm.at[idx0_v], vals0_v, semg)

    row0 = TC_ROWS + wid * R_W
    pltpu.sync_copy(tgt_hbm.at[pl.ds(row0, R_W)], dtgt_v)
    bufs = (buf0, buf1)
    sems = (sem0, sem1)

    def _grp(g, acc):
        r0 = row0 + g * 16
        tv = dtgt_v[pl.ds(pl.multiple_of(g * 16, 16), 16)]
        mvf = jnp.where(tv != PAD, 1.0, 0.0).astype(jnp.float32)
        mr = [mvf.at[jnp.full((L,), r, jnp.int32)].get(mode="promise_in_bounds")
              for r in range(16)]

        cps = [None, None]
        cps[0] = pltpu.async_copy(
            x_hbm.at[pl.ds(r0, 16), pl.ds(0, CW)], bufs[0], sems[0])
        for c in range(N_CHK):
            s = c & 1
            if c + 1 < N_CHK:
                cps[(c + 1) & 1] = pltpu.async_copy(
                    x_hbm.at[pl.ds(r0, 16), pl.ds((c + 1) * CW, CW)],
                    bufs[(c + 1) & 1], sems[(c + 1) & 1])
            cps[s].wait()
            buf = bufs[s]

            def _chunk(k, a):
                col = pl.multiple_of(k * L, L)
                for r in range(16):
                    a = a + buf[r, pl.ds(col, L)] * mr[r]
                return a

            acc = lax.fori_loop(0, CW // L, _chunk, acc)
        return acc

    accd = lax.fori_loop(0, N_GRP, _grp, jnp.zeros((L,), jnp.float32))

    gcp.wait()
    gcp0.wait()
    accg = jnp.zeros((L,), jnp.float32)
    accb = jnp.zeros((L,), jnp.float32)
    acck = jnp.zeros((L,), jnp.float32)
    for j in range(BPW // L):
        sl = pl.ds(j * L, L)
        valid = tgt_v[sl] != PAD
        accg = accg + jnp.where(valid, vals_v[sl], 0.0)
        accb = accb + jnp.where(valid, vals0_v[sl], 0.0)
        acck = acck + jnp.where(valid, 1.0, 0.0)

    res_v[0, :] = accg
    res_v[1, :] = accb
    res_v[2, :] = acck
    res_v[3, :] = accd
    pltpu.sync_copy(res_v, out_hbm.at[wid])


_sc_gather = pl.kernel(
    _sc_body,
    out_type=jax.ShapeDtypeStruct((NW, 4, L), jnp.float32),
    mesh=plsc.VectorSubcoreMesh(core_axis_name="c", subcore_axis_name="s"),
    scratch_types=[
        pltpu.VMEM((BPW,), jnp.int32),
        pltpu.VMEM((BPW,), jnp.int32),
        pltpu.VMEM((BPW,), jnp.int32),
        pltpu.VMEM((BPW,), jnp.float32),
        pltpu.VMEM((BPW,), jnp.float32),
        pltpu.VMEM((R_W,), jnp.int32),
        pltpu.VMEM((16, CW), jnp.float32),
        pltpu.VMEM((16, CW), jnp.float32),
        pltpu.VMEM((4, L), jnp.float32),
        pltpu.SemaphoreType.DMA,
        pltpu.SemaphoreType.DMA,
        pltpu.SemaphoreType.DMA,
    ],
)


def kernel(x, target):
    tgt = target.astype(jnp.int32)
    a_tc = _masked_sum(tgt.reshape(N_TOK, 1), x)[0, 0]
    res = _sc_gather(x, x.reshape(N_TOK * N_CLS), tgt)
    g = jnp.sum(res[:, 0, :])
    b = jnp.sum(res[:, 1, :])
    k = jnp.sum(res[:, 2, :])
    a = a_tc + jnp.sum(res[:, 3, :])
    return k * C0 - EPS * a + EPS * b + (EPS - CONF) * g


# speedup vs baseline: 2.6518x; 2.6518x over previous
import math

import jax
import jax.numpy as jnp
from jax import lax
from jax.experimental import pallas as pl
from jax.experimental.pallas import tpu as pltpu
from jax.experimental.pallas import tpu_sc as plsc

N_TOK = 4096
N_CLS = 32000
PAD = 0
SMOOTHING = 0.1
CONF = 1.0 - SMOOTHING
EPS = SMOOTHING / (N_CLS - 2)
C0 = (N_CLS - 2) * EPS * math.log(EPS) + CONF * math.log(CONF)

TC_ROWS = 2048
SC_ROWS = N_TOK - TC_ROWS

ROW_BLK = 128


def _tc_body(tgt_ref, x_ref, acc_ref):
    @pl.when(pl.program_id(0) == 0)
    def _():
        for q in range(4):
            acc_ref[0, q] = 0.0

    tgt = tgt_ref[...]
    m = tgt != PAD
    mf = m.astype(jnp.float32)
    xb = x_ref[...]
    col = lax.broadcasted_iota(jnp.int32, (ROW_BLK, N_CLS), 1)
    tgtm = jnp.where(m, tgt, -1)
    acc_ref[0, 0] += jnp.sum(xb * mf)
    acc_ref[0, 1] += jnp.sum(jnp.where(col == tgtm, xb, 0.0))
    acc_ref[0, 2] += jnp.sum(xb[:, 0:1] * mf)
    acc_ref[0, 3] += jnp.sum(mf)


_tc_pass = pl.pallas_call(
    _tc_body,
    grid=(TC_ROWS // ROW_BLK,),
    in_specs=[
        pl.BlockSpec((ROW_BLK, 1), lambda i: (i, 0)),
        pl.BlockSpec((ROW_BLK, N_CLS), lambda i: (i, 0)),
    ],
    out_specs=pl.BlockSpec((1, 4), lambda i: (0, 0), memory_space=pltpu.SMEM),
    out_shape=jax.ShapeDtypeStruct((1, 4), jnp.float32),
)

L = 16
NC, NS = 2, 16
NW = NC * NS
R_W = SC_ROWS // NW
N_GRP = R_W // 16
CW = 3200
N_CHK = N_CLS // CW


def _bcast(vec, r):
    return vec.at[jnp.full((L,), r, jnp.int32)].get(mode="promise_in_bounds")


def _sc_body(x_hbm, tgt_hbm, out_hbm, dtgt_v, buf0, buf1, res_v, sem0, sem1):
    wid = lax.axis_index("s") * NC + lax.axis_index("c")
    lane_ids = lax.iota(jnp.int32, L)
    row0 = TC_ROWS + wid * R_W
    pltpu.sync_copy(tgt_hbm.at[pl.ds(row0, R_W)], dtgt_v)

    acck = jnp.zeros((L,), jnp.float32)
    for q in range(R_W // L):
        acck = acck + jnp.where(dtgt_v[pl.ds(q * L, L)] != PAD, 1.0, 0.0)

    bufs = (buf0, buf1)
    sems = (sem0, sem1)
    lane0 = jnp.where(lane_ids == 0, 1.0, 0.0).astype(jnp.float32)

    def _grp(g, carry):
        accd, accg, accb = carry
        r0 = row0 + g * 16
        tv = dtgt_v[pl.ds(pl.multiple_of(g * 16, 16), 16)]
        tvm = jnp.where(tv != PAD, tv, -1)
        mvf = jnp.where(tv != PAD, 1.0, 0.0).astype(jnp.float32)
        mr = [_bcast(mvf, r) for r in range(16)]
        tr = [_bcast(tvm, r) for r in range(16)]

        cps = [None, None]
        cps[0] = pltpu.async_copy(
            x_hbm.at[pl.ds(r0, 16), pl.ds(0, CW)], bufs[0], sems[0])
        for c in range(N_CHK):
            s = c & 1
            if c + 1 < N_CHK:
                cps[(c + 1) & 1] = pltpu.async_copy(
                    x_hbm.at[pl.ds(r0, 16), pl.ds((c + 1) * CW, CW)],
                    bufs[(c + 1) & 1], sems[(c + 1) & 1])
            cps[s].wait()
            buf = bufs[s]

            def _chunk(k, a):
                ad, ag = a
                colb = pl.multiple_of(k * L, L)
                col = c * CW + colb + lane_ids
                for r in range(16):
                    v = buf[r, pl.ds(colb, L)]
                    ad = ad + v * mr[r]
                    ag = ag + jnp.where(col == tr[r], v, 0.0)
                return ad, ag

            accd, accg = lax.fori_loop(0, CW // L, _chunk, (accd, accg))
            if c == 0:
                for r in range(16):
                    accb = accb + buf[r, pl.ds(0, L)] * lane0 * mr[r]
        return accd, accg, accb

    zero = jnp.zeros((L,), jnp.float32)
    accd, accg, accb = lax.fori_loop(0, N_GRP, _grp, (zero, zero, zero))

    res_v[0, :] = accd
    res_v[1, :] = accg
    res_v[2, :] = accb
    res_v[3, :] = acck
    pltpu.sync_copy(res_v, out_hbm.at[wid])


_sc_pass = pl.kernel(
    _sc_body,
    out_type=jax.ShapeDtypeStruct((NW, 4, L), jnp.float32),
    mesh=plsc.VectorSubcoreMesh(core_axis_name="c", subcore_axis_name="s"),
    scratch_types=[
        pltpu.VMEM((R_W,), jnp.int32),
        pltpu.VMEM((16, CW), jnp.float32),
        pltpu.VMEM((16, CW), jnp.float32),
        pltpu.VMEM((4, L), jnp.float32),
        pltpu.SemaphoreType.DMA,
        pltpu.SemaphoreType.DMA,
    ],
)


def kernel(x, target):
    tgt = target.astype(jnp.int32)
    tc = _tc_pass(tgt.reshape(N_TOK, 1), x)
    res = _sc_pass(x, tgt)
    a = tc[0, 0] + jnp.sum(res[:, 0, :])
    g = tc[0, 1] + jnp.sum(res[:, 1, :])
    b = tc[0, 2] + jnp.sum(res[:, 2, :])
    k = tc[0, 3] + jnp.sum(res[:, 3, :])
    return k * C0 - EPS * a + EPS * b + (EPS - CONF) * g
